# baseline (speedup 1.0000x reference)
import jax
import jax.numpy as jnp
from jax import lax
from jax.experimental import pallas as pl
from jax.experimental.pallas import tpu as pltpu
from jax.experimental.pallas import tpu_sc as plsc

VOCAB = 100000
D = 128
SCALE = float(D) ** 0.5

NC = 2
NS = 16
NW = NC * NS

B_TOTAL = 1024 * 200
B_PER_W = B_TOTAL // NW
CHUNK = 128
NCHUNK = B_PER_W // CHUNK


NBUF = 6
INFLIGHT = 3


def _emb_body(idx_hbm, table_hbm, out_hbm, *scratch):
  idx_v = scratch[0]
  bufs = scratch[1:1 + NBUF]
  gsems = scratch[1 + NBUF:1 + 2 * NBUF]
  ssems = scratch[1 + 2 * NBUF:1 + 3 * NBUF]

  wid = lax.axis_index("s") * NC + lax.axis_index("c")
  base = wid * B_PER_W

  pltpu.sync_copy(idx_hbm.at[pl.ds(base, B_PER_W)], idx_v)

  def start_gather(c, phase):
    pltpu.make_async_copy(
        table_hbm.at[idx_v.at[pl.ds(c * CHUNK, CHUNK)]],
        bufs[phase],
        gsems[phase],
    ).start()

  def wait_scatter(phase):
    pltpu.make_async_copy(bufs[phase], out_hbm.at[pl.ds(base, CHUNK)],
                          ssems[phase]).wait()

  def do_chunk(cc, phase):
    buf = bufs[phase]
    pltpu.make_async_copy(table_hbm.at[idx_v.at[pl.ds(0, CHUNK)]], buf,
                          gsems[phase]).wait()

    @pl.loop(0, CHUNK)
    def _scale_row(r):
      for j in range(8):
        sl = (r, pl.ds(j * 16, 16))
        buf[sl] = buf[sl] * SCALE

    pltpu.make_async_copy(buf, out_hbm.at[pl.ds(base + cc * CHUNK, CHUNK)],
                          ssems[phase]).start()

    p2 = (phase + INFLIGHT) % NBUF
    if isinstance(cc, int):
      if cc + INFLIGHT < NCHUNK:
        if cc >= INFLIGHT:
          wait_scatter(p2)
        start_gather(cc + INFLIGHT, p2)
    else:
      @pl.when(cc >= INFLIGHT)
      def _():
        wait_scatter(p2)

      start_gather(cc + INFLIGHT, p2)

  for b in range(INFLIGHT):
    start_gather(b, b)

  MAIN = ((NCHUNK - INFLIGHT) // NBUF) * NBUF

  @pl.loop(0, MAIN, step=NBUF)
  def _chunk_loop(c):
    for phase in range(NBUF):
      do_chunk(c + phase, phase)

  for cc in range(MAIN, NCHUNK):
    do_chunk(cc, cc % NBUF)

  for cc in range(NCHUNK - NBUF, NCHUNK):
    wait_scatter(cc % NBUF)


@jax.jit
def _emb_call(x_flat, table):
  mesh = plsc.VectorSubcoreMesh(
      core_axis_name="c", subcore_axis_name="s", num_cores=NC,
      num_subcores=NS)
  return pl.kernel(
      _emb_body,
      out_type=jax.ShapeDtypeStruct((B_TOTAL, D), jnp.float32),
      mesh=mesh,
      scratch_types=(
          [pltpu.VMEM((B_PER_W,), jnp.int32)]
          + [pltpu.VMEM((CHUNK, D), jnp.float32)] * NBUF
          + [pltpu.SemaphoreType.DMA] * (2 * NBUF)
      ),
  )(x_flat, table)


def kernel(x, table):
  x_flat = x.reshape(-1).astype(jnp.int32)
  out = _emb_call(x_flat, table)
  return out.reshape(x.shape + (D,))
